# baseline (speedup 1.0000x reference)
import functools

import jax
import jax.numpy as jnp
from jax import lax
from jax.experimental import pallas as pl
from jax.experimental.pallas import tpu as pltpu
from jax.experimental.pallas import tpu_sc as plsc

_NN = {'p': 20000, 's': 20000, 'g': 100000}
_ETYPES = {
    'pg': ('p', 'g', 128000), 'ps': ('p', 's', 64000), 'sp': ('s', 'p', 64000),
    'sg': ('s', 'g', 128000), 'gp': ('g', 'p', 64000), 'gs': ('g', 's', 64000),
    'gg': ('g', 'g', 160000)}
_TKEYS = list(_ETYPES)

_B = 128
_DUMP = 64
_ZB = 40
_NTILE = 16
_CH = 4


def _part_count(n_dst, d):
    p = 2
    while (n_dst // p) * d * 4 > 6_400_000:
        p *= 2
    return p


def _build_passes(d, with_cnt):
    e_off = {}
    off = 0
    for k in _TKEYS:
        e_off[k] = off
        off += _ETYPES[k][2]
    e_tot = off
    out_off = {}
    off = 0
    for k in _TKEYS:
        out_off[k] = off
        off += _NN[_ETYPES[k][1]]
    sum_rows = off

    passes = []
    for k in _TKEYS:
        _, dst_t, e = _ETYPES[k]
        n_dst = _NN[dst_t]
        p_cnt = _part_count(n_dst, d)
        for j in range(p_cnt // 2):
            passes.append([e_off[k], e // _B, j, p_cnt // 2,
                           n_dst // p_cnt, out_off[k]])
    if with_cnt:
        for k in _TKEYS:
            _, dst_t, e = _ETYPES[k]
            n_dst = _NN[dst_t]
            p_cnt = _part_count(n_dst, d)
            for j in range(p_cnt // 2):
                passes.append([e_tot + e_off[k], e // _B, j, p_cnt // 2,
                               n_dst // p_cnt, sum_rows + out_off[k]])
    part_max = max(p[4] for p in passes)
    out_rows = sum_rows * (2 if with_cnt else 1)
    return passes, part_max, out_rows, e_tot, out_off, sum_rows


def _segsum_body(passes, d, *refs):
    (xcat, rows, cols, zref, out_ref,
     table, rowflat, colflat, idxbuf, rowsb, xrow, xcol, zbuf,
     gsem, xsem) = refs

    core = lax.axis_index("c")
    sub = lax.axis_index("s")

    pltpu.sync_copy(zref, zbuf)

    def munge(cref, b_off, base, part_rows):
        for i in range(_B // 16):
            c = cref[pl.ds(b_off + i * 16, 16)]
            inpart = (c >= base) & (c < base + part_rows)
            idx = jnp.where(inpart, c - base,
                            part_rows + (c & (_DUMP - 1)))
            idxbuf[0, pl.ds(i * 16, 16)] = idx

    def scat(ridref, base, part_rows, cref, coff):
        munge(cref, coff, base, part_rows)
        pltpu.async_copy(xcat.at[ridref], rowsb.at[0], xsem).wait()
        pltpu.sync_copy(rowsb.at[0], table.at[idxbuf.at[0]], add=True)

    def pass_body(e_base, nblk, jpart, npasses, part_rows, out_base):
        base = (core * npasses + jpart) * part_rows
        q = nblk // _NTILE
        r = nblk - q * _NTILE

        nzb = (part_rows + _DUMP) // _ZB + 2
        nzi = nzb // _NTILE + jnp.where(sub < nzb - (nzb // _NTILE) * _NTILE,
                                        1, 0)

        def zbody(i, c2):
            z = sub + i * _NTILE
            pltpu.sync_copy(zbuf.at[pl.ds(0, _ZB)],
                            table.at[pl.ds(z * _ZB, _ZB)])
            return c2
        lax.fori_loop(0, nzi, zbody, 0)
        plsc.subcore_barrier()

        start_blk = sub * q + jnp.minimum(sub, r)
        start_e = e_base + start_blk * _B

        nfull = q // _CH

        def chunk_body(ch, c2):
            b0e = start_e + ch * (_CH * _B)
            pltpu.sync_copy(cols.at[pl.ds(b0e, _CH * _B)], colflat)
            pltpu.sync_copy(rows.at[pl.ds(b0e, _CH * _B)], rowflat)
            pltpu.async_copy(xcat.at[rowflat.at[pl.ds(0, _B)]],
                             rowsb.at[0], gsem.at[0])
            for i in range(_CH):
                sl = i % 2
                if i + 1 < _CH:
                    pltpu.async_copy(
                        xcat.at[rowflat.at[pl.ds((i + 1) * _B, _B)]],
                        rowsb.at[1 - sl], gsem.at[1 - sl])
                munge(colflat, i * _B, base, part_rows)
                pltpu.make_async_copy(
                    xcat.at[rowflat.at[pl.ds(i * _B, _B)]],
                    rowsb.at[sl], gsem.at[sl]).wait()
                pltpu.sync_copy(rowsb.at[sl], table.at[idxbuf.at[0]],
                                add=True)
            return c2
        lax.fori_loop(0, nfull, chunk_body, 0)

        def tail_body(tb, c2):
            tbe = start_e + (nfull * _CH + tb) * _B
            pltpu.sync_copy(cols.at[pl.ds(tbe, _B)], xcol)
            pltpu.sync_copy(rows.at[pl.ds(tbe, _B)], xrow)
            scat(xrow, base, part_rows, xcol, 0)
            return c2
        lax.fori_loop(0, q - nfull * _CH, tail_body, 0)

        @pl.when(sub < r)
        def _():
            xbe = e_base + (start_blk + q) * _B
            pltpu.sync_copy(cols.at[pl.ds(xbe, _B)], xcol)
            pltpu.sync_copy(rows.at[pl.ds(xbe, _B)], xrow)
            scat(xrow, base, part_rows, xcol, 0)

        plsc.subcore_barrier()

        nwb = part_rows // _ZB
        nwi = nwb // _NTILE + jnp.where(sub < nwb - (nwb // _NTILE) * _NTILE,
                                        1, 0)

        def wbody(i, c2):
            z = sub + i * _NTILE
            pltpu.sync_copy(
                table.at[pl.ds(z * _ZB, _ZB)],
                out_ref.at[pl.ds(out_base + base + z * _ZB, _ZB)])
            return c2
        lax.fori_loop(0, nwi, wbody, 0)
        plsc.subcore_barrier()

    for p in passes:
        pass_body(*p)


def _make_segsum(d, with_cnt):
    passes, part_max, out_rows, e_tot, _, _ = _build_passes(d, with_cnt)
    out_type = jax.ShapeDtypeStruct((out_rows, d), jnp.float32)

    scratch = [
        pltpu.VMEM_SHARED((part_max + _DUMP + 2 * _ZB, d), jnp.float32),
        pltpu.VMEM((_CH * _B,), jnp.int32),
        pltpu.VMEM((_CH * _B,), jnp.int32),
        pltpu.VMEM((1, _B), jnp.int32),
        pltpu.VMEM((2, _B, d), jnp.float32),
        pltpu.VMEM((_B,), jnp.int32),
        pltpu.VMEM((_B,), jnp.int32),
        pltpu.VMEM((_ZB, d), jnp.float32),
        pltpu.SemaphoreType.DMA((2,)),
        pltpu.SemaphoreType.DMA,
    ]
    body = functools.partial(_segsum_body, passes, d)
    return pl.kernel(
        body, out_type=out_type,
        mesh=plsc.VectorSubcoreMesh(core_axis_name="c", subcore_axis_name="s"),
        scratch_types=scratch,
        compiler_params=pltpu.CompilerParams(use_tc_tiling_on_sc=False))


def _segsums(xd, eis, d, with_cnt):
    passes, _, _, e_tot, out_off, sum_rows = _build_passes(d, with_cnt)
    fn = _make_segsum(d, with_cnt)

    n_p, n_s = _NN['p'], _NN['s']
    src_off = {'p': 0, 's': n_p, 'g': n_p + n_s}
    ones_row = n_p + n_s + _NN['g']
    xcat = jnp.concatenate(
        [xd['p'], xd['s'], xd['g'],
         jnp.ones((8, d), jnp.float32)], axis=0)

    rows_l = [eis[k][0] + src_off[_ETYPES[k][0]] for k in _TKEYS]
    cols_l = [eis[k][1] for k in _TKEYS]
    rows_cat = jnp.concatenate(rows_l)
    cols_cat = jnp.concatenate(cols_l)
    if with_cnt:
        rows_cat = jnp.concatenate(
            [rows_cat, jnp.full((e_tot,), ones_row, jnp.int32)])
        cols_cat = jnp.concatenate([cols_cat, cols_cat])

    zeros = jnp.zeros((_ZB, d), jnp.float32)

    big = fn(xcat, rows_cat, cols_cat, zeros)
    sums = {k: lax.dynamic_slice_in_dim(big, out_off[k],
                                        _NN[_ETYPES[k][1]], 0)
            for k in _TKEYS}
    cnts = None
    if with_cnt:
        cnts = {k: lax.dynamic_slice_in_dim(big, sum_rows + out_off[k],
                                            _NN[_ETYPES[k][1]], 0)[:, 0]
                for k in _TKEYS}
    return sums, cnts


def _linear(x, p):
    return x @ p['w'] + p['b']


def _bn(x, p, eps=1e-5):
    mu = jnp.mean(x, 0)
    var = jnp.var(x, 0)
    return (x - mu) / jnp.sqrt(var + eps) * p['g'] + p['b']


def _sage_edge_dense(s_sum, cnt, x_dst, win_mask, ea_w, p):
    o = p['sage']['wl'].shape[1]
    agg = s_sum / cnt[:, None]
    out = agg @ p['sage']['wl'] + p['sage']['bl'] + x_dst @ p['sage']['wr']
    emb = jax.nn.relu(_linear(ea_w, p['emb']))
    t_emb = _linear(emb, p['temb'])
    t_attr = _linear(ea_w, p['tattr'])
    t = t_emb + t_attr
    wg = p['gate']['w']
    g0 = t @ wg[o:2 * o] + t_attr @ wg[2 * o:] + p['gate']['b']
    gate = jax.nn.sigmoid(out @ wg[:o] + g0)
    out = out + win_mask[:, None] * (gate * t)
    out = _bn(out, p['bn'])
    out = out + out
    return jax.nn.relu(out)


def _hetero(xd, sums, cnts, meta, pl_):
    outs = {'p': [], 's': [], 'g': []}
    for k, (s, t_dst, _) in _ETYPES.items():
        win_mask, ea_w = meta[k]
        outs[t_dst].append(_sage_edge_dense(sums[k], cnts[k], xd[t_dst],
                                            win_mask, ea_w, pl_[k]))
    for t in ['p', 's', 'g']:
        sp = pl_['self_' + t]
        outs[t].append(xd[t] @ (sp['wl'] + sp['wr']) + sp['bl'])
    return {t: sum(outs[t]) for t in outs}


def _final_lin_kernel(x_ref, w_ref, b_ref, a_ref, o_ref):
    y = x_ref[...] @ w_ref[...] + b_ref[0, 0]
    a = a_ref[0, 0]
    o_ref[...] = jnp.where(y >= 0, y, a * y)


def _final_lin(x, w, b, a):
    n = x.shape[0]
    blk = 2000
    return pl.pallas_call(
        _final_lin_kernel,
        grid=(n // blk,),
        in_specs=[
            pl.BlockSpec((blk, x.shape[1]), lambda i: (i, 0)),
            pl.BlockSpec((x.shape[1], 1), lambda i: (0, 0)),
            pl.BlockSpec((1, 1), lambda i: (0, 0)),
            pl.BlockSpec((1, 1), lambda i: (0, 0)),
        ],
        out_specs=pl.BlockSpec((blk, 1), lambda i: (i, 0)),
        out_shape=jax.ShapeDtypeStruct((n, 1), jnp.float32),
    )(x, w, b.reshape(1, 1), a.reshape(1, 1))


def kernel(x_pfas, x_sw, x_gw, eas, params, eis):
    xd = {'p': x_pfas, 's': x_sw, 'g': x_gw}
    nn = {t: v.shape[0] for t, v in xd.items()}
    xd = {t: jax.nn.relu(_bn(_linear(x, params['node_red'][t]),
                             params['node_bn'][t]))
          for t, x in xd.items()}
    ead = {k: jax.nn.relu(_bn(_linear(eas[k], params['edge_red'][k]),
                              params['edge_bn'][k]))
           for k in _ETYPES}

    meta = {}
    for k, (s, d, _) in _ETYPES.items():
        col = eis[k][1]
        n = nn[d]
        e = col.shape[0]
        win = jax.ops.segment_max(jnp.arange(e, dtype=jnp.int32), col,
                                  num_segments=n)
        mask = (win >= 0) & (win < e)
        winc = jnp.where(mask, win, 0)
        meta[k] = (mask.astype(jnp.float32), ead[k][winc])

    sums1, cnts = _segsums(xd, eis, 32, True)
    cnts = {k: jnp.maximum(v, 1.0) for k, v in cnts.items()}
    xd = _hetero(xd, sums1, cnts, meta, params['conv1'])
    xd = {t: jax.nn.relu(v) for t, v in xd.items()}

    sums2, _ = _segsums(xd, eis, 64, False)
    xd = _hetero(xd, sums2, cnts, meta, params['conv2'])
    xd = {t: jax.nn.relu(v) for t, v in xd.items()}

    w, b, a = params['lin']['w'], params['lin']['b'], params['prelu']
    gw = _final_lin(xd['g'], w, b, a)
    sw = _final_lin(xd['s'], w, b, a)
    return gw, sw, xd['p']


# speedup vs baseline: 2.2071x; 2.2071x over previous
import functools

import jax
import jax.numpy as jnp
from jax import lax
from jax.experimental import pallas as pl
from jax.experimental.pallas import tpu as pltpu
from jax.experimental.pallas import tpu_sc as plsc

_NN = {'p': 20000, 's': 20000, 'g': 100000}
_ETYPES = {
    'pg': ('p', 'g', 128000), 'ps': ('p', 's', 64000), 'sp': ('s', 'p', 64000),
    'sg': ('s', 'g', 128000), 'gp': ('g', 'p', 64000), 'gs': ('g', 's', 64000),
    'gg': ('g', 'g', 160000)}
_TKEYS = list(_ETYPES)

_B = 128
_DUMP = 64
_ZB = 40
_NTILE = 16
_CH = 4


def _part_count(n_dst, d):
    p = 2
    while (n_dst // p) * d * 4 > 6_400_000:
        p *= 2
    return p


def _build_passes(d, with_cnt):
    e_off = {}
    off = 0
    for k in _TKEYS:
        e_off[k] = off
        off += _ETYPES[k][2]
    e_tot = off
    out_off = {}
    off = 0
    for k in _TKEYS:
        out_off[k] = off
        off += _NN[_ETYPES[k][1]]
    sum_rows = off

    passes = []
    for k in _TKEYS:
        _, dst_t, e = _ETYPES[k]
        n_dst = _NN[dst_t]
        p_cnt = _part_count(n_dst, d)
        for j in range(p_cnt // 2):
            passes.append([e_off[k], e // _B, j, p_cnt // 2,
                           n_dst // p_cnt, out_off[k], True])
    if with_cnt:
        for k in _TKEYS:
            _, dst_t, e = _ETYPES[k]
            n_dst = _NN[dst_t]
            p_cnt = _part_count(n_dst, d)
            for j in range(p_cnt // 2):
                passes.append([e_off[k], e // _B, j, p_cnt // 2,
                               n_dst // p_cnt, sum_rows + out_off[k], False])
    part_max = max(p[4] for p in passes)
    out_rows = sum_rows * (2 if with_cnt else 1)
    return passes, part_max, out_rows, e_tot, out_off, sum_rows


def _segsum_body(passes, d, with_cnt, *refs):
    if with_cnt:
        (xcat, rows, cols, zref, oref, out_ref,
         table, rowflat, colflat, idxbuf, rowsb, xrow, xcol, zbuf, onesb,
         gsem, xsem) = refs
    else:
        (xcat, rows, cols, zref, out_ref,
         table, rowflat, colflat, idxbuf, rowsb, xrow, xcol, zbuf, onesb,
         gsem, xsem) = refs

    core = lax.axis_index("c")
    sub = lax.axis_index("s")

    pltpu.sync_copy(zref, zbuf)
    if with_cnt:
        pltpu.sync_copy(oref, onesb)

    def munge(cref, b_off, base, part_rows):
        for i in range(_B // 16):
            c = cref[pl.ds(b_off + i * 16, 16)]
            inpart = (c >= base) & (c < base + part_rows)
            idx = jnp.where(inpart, c - base,
                            part_rows + (c & (_DUMP - 1)))
            idxbuf[0, pl.ds(i * 16, 16)] = idx

    def scat(ridref, base, part_rows, cref, coff):
        munge(cref, coff, base, part_rows)
        pltpu.async_copy(xcat.at[ridref], rowsb.at[0], xsem).wait()
        pltpu.sync_copy(rowsb.at[0], table.at[idxbuf.at[0]], add=True)

    def pass_body(e_base, nblk, jpart, npasses, part_rows, out_base, gather):
        base = (core * npasses + jpart) * part_rows
        q = nblk // _NTILE
        r = nblk - q * _NTILE

        nzb = (part_rows + _DUMP) // _ZB + 2
        nzi = nzb // _NTILE + jnp.where(sub < nzb - (nzb // _NTILE) * _NTILE,
                                        1, 0)

        def zbody(i, c2):
            z = sub + i * _NTILE
            pltpu.sync_copy(zbuf.at[pl.ds(0, _ZB)],
                            table.at[pl.ds(z * _ZB, _ZB)])
            return c2
        lax.fori_loop(0, nzi, zbody, 0)
        plsc.subcore_barrier()

        start_blk = sub * q + jnp.minimum(sub, r)
        start_e = e_base + start_blk * _B

        nfull = q // _CH

        def chunk_body(ch, c2):
            b0e = start_e + ch * (_CH * _B)
            pltpu.sync_copy(cols.at[pl.ds(b0e, _CH * _B)], colflat)
            if gather:
                pltpu.sync_copy(rows.at[pl.ds(b0e, _CH * _B)], rowflat)
                pltpu.async_copy(xcat.at[rowflat.at[pl.ds(0, _B)]],
                                 rowsb.at[0], gsem.at[0])
                for i in range(_CH):
                    sl = i % 2
                    if i + 1 < _CH:
                        pltpu.async_copy(
                            xcat.at[rowflat.at[pl.ds((i + 1) * _B, _B)]],
                            rowsb.at[1 - sl], gsem.at[1 - sl])
                    munge(colflat, i * _B, base, part_rows)
                    pltpu.make_async_copy(
                        xcat.at[rowflat.at[pl.ds(i * _B, _B)]],
                        rowsb.at[sl], gsem.at[sl]).wait()
                    pltpu.sync_copy(rowsb.at[sl], table.at[idxbuf.at[0]],
                                    add=True)
            else:
                for i in range(_CH):
                    munge(colflat, i * _B, base, part_rows)
                    pltpu.sync_copy(onesb, table.at[idxbuf.at[0]],
                                    add=True)
            return c2
        lax.fori_loop(0, nfull, chunk_body, 0)

        def tail_body(tb, c2):
            tbe = start_e + (nfull * _CH + tb) * _B
            pltpu.sync_copy(cols.at[pl.ds(tbe, _B)], xcol)
            if gather:
                pltpu.sync_copy(rows.at[pl.ds(tbe, _B)], xrow)
                scat(xrow, base, part_rows, xcol, 0)
            else:
                munge(xcol, 0, base, part_rows)
                pltpu.sync_copy(onesb, table.at[idxbuf.at[0]], add=True)
            return c2
        lax.fori_loop(0, q - nfull * _CH, tail_body, 0)

        @pl.when(sub < r)
        def _():
            xbe = e_base + (start_blk + q) * _B
            pltpu.sync_copy(cols.at[pl.ds(xbe, _B)], xcol)
            if gather:
                pltpu.sync_copy(rows.at[pl.ds(xbe, _B)], xrow)
                scat(xrow, base, part_rows, xcol, 0)
            else:
                munge(xcol, 0, base, part_rows)
                pltpu.sync_copy(onesb, table.at[idxbuf.at[0]], add=True)

        plsc.subcore_barrier()

        nwb = part_rows // _ZB
        nwi = nwb // _NTILE + jnp.where(sub < nwb - (nwb // _NTILE) * _NTILE,
                                        1, 0)

        def wbody(i, c2):
            z = sub + i * _NTILE
            pltpu.sync_copy(
                table.at[pl.ds(z * _ZB, _ZB)],
                out_ref.at[pl.ds(out_base + base + z * _ZB, _ZB)])
            return c2
        lax.fori_loop(0, nwi, wbody, 0)
        plsc.subcore_barrier()

    for p in passes:
        pass_body(*p)


def _make_segsum(d, with_cnt):
    passes, part_max, out_rows, e_tot, _, _ = _build_passes(d, with_cnt)
    out_type = jax.ShapeDtypeStruct((out_rows, d), jnp.float32)

    scratch = [
        pltpu.VMEM_SHARED((part_max + _DUMP + 2 * _ZB, d), jnp.float32),
        pltpu.VMEM((_CH * _B,), jnp.int32),
        pltpu.VMEM((_CH * _B,), jnp.int32),
        pltpu.VMEM((1, _B), jnp.int32),
        pltpu.VMEM((2, _B, d), jnp.float32),
        pltpu.VMEM((_B,), jnp.int32),
        pltpu.VMEM((_B,), jnp.int32),
        pltpu.VMEM((_ZB, d), jnp.float32),
        pltpu.VMEM((_B, d) if with_cnt else (8, d), jnp.float32),
        pltpu.SemaphoreType.DMA((2,)),
        pltpu.SemaphoreType.DMA,
    ]
    body = functools.partial(_segsum_body, passes, d, with_cnt)
    return pl.kernel(
        body, out_type=out_type,
        mesh=plsc.VectorSubcoreMesh(core_axis_name="c", subcore_axis_name="s"),
        scratch_types=scratch,
        compiler_params=pltpu.CompilerParams(use_tc_tiling_on_sc=False))


def _segsums(xd, eis, d, with_cnt):
    passes, _, _, e_tot, out_off, sum_rows = _build_passes(d, with_cnt)
    fn = _make_segsum(d, with_cnt)

    n_p, n_s = _NN['p'], _NN['s']
    src_off = {'p': 0, 's': n_p, 'g': n_p + n_s}
    xcat = jnp.concatenate([xd['p'], xd['s'], xd['g']], axis=0)

    rows_cat = jnp.concatenate(
        [eis[k][0] + src_off[_ETYPES[k][0]] for k in _TKEYS])
    cols_cat = jnp.concatenate([eis[k][1] for k in _TKEYS])

    zeros = jnp.zeros((_ZB, d), jnp.float32)
    args = [xcat, rows_cat, cols_cat, zeros]
    if with_cnt:
        args.append(jnp.ones((_B, d), jnp.float32))

    big = fn(*args)
    sums = {k: lax.dynamic_slice_in_dim(big, out_off[k],
                                        _NN[_ETYPES[k][1]], 0)
            for k in _TKEYS}
    cnts = None
    if with_cnt:
        cnts = {k: lax.dynamic_slice_in_dim(big, sum_rows + out_off[k],
                                            _NN[_ETYPES[k][1]], 0)[:, 0]
                for k in _TKEYS}
    return sums, cnts


def _linear(x, p):
    return x @ p['w'] + p['b']


def _bn(x, p, eps=1e-5):
    mu = jnp.mean(x, 0)
    var = jnp.var(x, 0)
    return (x - mu) / jnp.sqrt(var + eps) * p['g'] + p['b']


def _sage_edge_dense(s_sum, cnt, x_dst, win_mask, ea_w, p):
    o = p['sage']['wl'].shape[1]
    agg = s_sum / cnt[:, None]
    out = agg @ p['sage']['wl'] + p['sage']['bl'] + x_dst @ p['sage']['wr']
    emb = jax.nn.relu(_linear(ea_w, p['emb']))
    t_emb = _linear(emb, p['temb'])
    t_attr = _linear(ea_w, p['tattr'])
    t = t_emb + t_attr
    wg = p['gate']['w']
    g0 = t @ wg[o:2 * o] + t_attr @ wg[2 * o:] + p['gate']['b']
    gate = jax.nn.sigmoid(out @ wg[:o] + g0)
    out = out + win_mask[:, None] * (gate * t)
    out = _bn(out, p['bn'])
    out = out + out
    return jax.nn.relu(out)


def _hetero(xd, sums, cnts, meta, pl_):
    outs = {'p': [], 's': [], 'g': []}
    for k, (s, t_dst, _) in _ETYPES.items():
        win_mask, ea_w = meta[k]
        outs[t_dst].append(_sage_edge_dense(sums[k], cnts[k], xd[t_dst],
                                            win_mask, ea_w, pl_[k]))
    for t in ['p', 's', 'g']:
        sp = pl_['self_' + t]
        outs[t].append(xd[t] @ (sp['wl'] + sp['wr']) + sp['bl'])
    return {t: sum(outs[t]) for t in outs}


def _final_lin_kernel(x_ref, w_ref, b_ref, a_ref, o_ref):
    y = x_ref[...] @ w_ref[...] + b_ref[0, 0]
    a = a_ref[0, 0]
    o_ref[...] = jnp.where(y >= 0, y, a * y)


def _final_lin(x, w, b, a):
    n = x.shape[0]
    blk = 2000
    return pl.pallas_call(
        _final_lin_kernel,
        grid=(n // blk,),
        in_specs=[
            pl.BlockSpec((blk, x.shape[1]), lambda i: (i, 0)),
            pl.BlockSpec((x.shape[1], 1), lambda i: (0, 0)),
            pl.BlockSpec((1, 1), lambda i: (0, 0)),
            pl.BlockSpec((1, 1), lambda i: (0, 0)),
        ],
        out_specs=pl.BlockSpec((blk, 1), lambda i: (i, 0)),
        out_shape=jax.ShapeDtypeStruct((n, 1), jnp.float32),
    )(x, w, b.reshape(1, 1), a.reshape(1, 1))


def kernel(x_pfas, x_sw, x_gw, eas, params, eis):
    xd = {'p': x_pfas, 's': x_sw, 'g': x_gw}
    nn = {t: v.shape[0] for t, v in xd.items()}
    xd = {t: jax.nn.relu(_bn(_linear(x, params['node_red'][t]),
                             params['node_bn'][t]))
          for t, x in xd.items()}
    ead = {k: jax.nn.relu(_bn(_linear(eas[k], params['edge_red'][k]),
                              params['edge_bn'][k]))
           for k in _ETYPES}

    meta = {}
    for k, (s, d, _) in _ETYPES.items():
        col = eis[k][1]
        n = nn[d]
        e = col.shape[0]
        win = jax.ops.segment_max(jnp.arange(e, dtype=jnp.int32), col,
                                  num_segments=n)
        mask = (win >= 0) & (win < e)
        winc = jnp.where(mask, win, 0)
        meta[k] = (mask.astype(jnp.float32), ead[k][winc])

    sums1, cnts = _segsums(xd, eis, 32, True)
    cnts = {k: jnp.maximum(v, 1.0) for k, v in cnts.items()}
    xd = _hetero(xd, sums1, cnts, meta, params['conv1'])
    xd = {t: jax.nn.relu(v) for t, v in xd.items()}

    sums2, _ = _segsums(xd, eis, 64, False)
    xd = _hetero(xd, sums2, cnts, meta, params['conv2'])
    xd = {t: jax.nn.relu(v) for t, v in xd.items()}

    w, b, a = params['lin']['w'], params['lin']['b'], params['prelu']
    gw = _final_lin(xd['g'], w, b, a)
    sw = _final_lin(xd['s'], w, b, a)
    return gw, sw, xd['p']
